# baseline (speedup 1.0000x reference)
import functools

import jax
import jax.numpy as jnp
from jax import lax
from jax.experimental import pallas as pl
from jax.experimental.pallas import tpu as pltpu
from jax.experimental.pallas import tpu_sc as plsc

LANES = 16
N_TILES = 16
N_CORES = 2
CHUNK = 128


def _mm_body(x_ref, w_ref, lo_ref, hi_ref):
    s = jnp.dot(x_ref[...], w_ref[...], preferred_element_type=jnp.float32)
    h = s.shape[1] // 2
    lo_ref[...] = s[:, :h]
    hi_ref[...] = s[:, h:]


def _matmul_halves(x, W):
    n, f = x.shape
    o = W.shape[1]
    h = o // 2
    blk = 1000
    grid = (n // blk,)
    return pl.pallas_call(
        _mm_body,
        grid=grid,
        in_specs=[
            pl.BlockSpec((blk, f), lambda i: (i, 0)),
            pl.BlockSpec((f, o), lambda i: (0, 0)),
        ],
        out_specs=[
            pl.BlockSpec((blk, h), lambda i: (i, 0)),
            pl.BlockSpec((blk, h), lambda i: (i, 0)),
        ],
        out_shape=[
            jax.ShapeDtypeStruct((n, h), jnp.float32),
            jax.ShapeDtypeStruct((n, h), jnp.float32),
        ],
    )(x, W)


def _make_sc_agg(n_nodes, half, n_chunks):
    rows_per_tile = n_nodes // N_TILES
    wb_chunk = CHUNK
    n_wb = rows_per_tile // wb_chunk
    mesh = plsc.VectorSubcoreMesh(core_axis_name="c", subcore_axis_name="s",
                                  num_cores=N_CORES, num_subcores=N_TILES)

    @functools.partial(
        pl.kernel,
        out_type=[
            jax.ShapeDtypeStruct((n_nodes, half), jnp.float32),
            jax.ShapeDtypeStruct((n_nodes, half), jnp.float32),
        ],
        mesh=mesh,
        scratch_types=[
            pltpu.VMEM((n_chunks, CHUNK), jnp.int32),
            pltpu.VMEM((n_chunks, CHUNK), jnp.int32),
            pltpu.VMEM((n_chunks, CHUNK), jnp.float32),
            pltpu.VMEM((CHUNK, half), jnp.float32),
            pltpu.VMEM_SHARED((n_nodes, half), jnp.float32),
            pltpu.SemaphoreType.DMA,
        ],
    )
    def sc_agg(src_hbm, dst_hbm, w_hbm, lo_hbm, hi_hbm, out_lo, out_hi,
               src_v, dst_v, w_v, rows, acc, sem):
        c = lax.axis_index("c")
        t = lax.axis_index("s")

        pltpu.sync_copy(src_hbm.at[t], src_v)
        pltpu.sync_copy(dst_hbm.at[t], dst_v)
        pltpu.sync_copy(w_hbm.at[t], w_v)

        def _zero_row(r, carry):
            for j in range(half // LANES):
                rows[r, pl.ds(j * LANES, LANES)] = jnp.zeros(
                    (LANES,), jnp.float32)
            return carry
        lax.fori_loop(0, wb_chunk, _zero_row, 0)
        for q in range(n_wb):
            pltpu.sync_copy(
                rows.at[pl.ds(0, wb_chunk)],
                acc.at[pl.ds(t * rows_per_tile + q * wb_chunk, wb_chunk)])
        plsc.subcore_barrier()

        def _pipeline(sup_hbm):
            def _chunk(ci, carry):
                pltpu.async_copy(sup_hbm.at[src_v.at[ci]], rows, sem).wait()

                def _scale16(g, inner):
                    base = g * LANES
                    wvec = w_v[ci, pl.ds(base, LANES)]
                    for lane in range(LANES):
                        wv = wvec[lane]
                        for j in range(half // LANES):
                            sl = pl.ds(j * LANES, LANES)
                            rows[base + lane, sl] = rows[base + lane, sl] * wv
                    return inner
                lax.fori_loop(0, CHUNK // LANES, _scale16, 0)
                pltpu.sync_copy(rows, acc.at[dst_v.at[ci]], add=True)
                return carry
            lax.fori_loop(0, n_chunks, _chunk, 0)

        pl.when(c == 0)(lambda: _pipeline(lo_hbm))
        pl.when(c == 1)(lambda: _pipeline(hi_hbm))
        plsc.subcore_barrier()

        def _writeback(out_hbm):
            for q in range(n_wb):
                row0 = t * rows_per_tile + q * wb_chunk
                pltpu.sync_copy(acc.at[pl.ds(row0, wb_chunk)],
                                rows.at[pl.ds(0, wb_chunk)])
                pltpu.sync_copy(rows.at[pl.ds(0, wb_chunk)],
                                out_hbm.at[pl.ds(row0, wb_chunk)])

        pl.when(c == 0)(lambda: _writeback(out_lo))
        pl.when(c == 1)(lambda: _writeback(out_hi))

    return sc_agg


def kernel(x, edge_index, edge_weight, W, b):
    n_nodes = x.shape[0]
    n_edges = edge_weight.shape[0]
    half = W.shape[1] // 2

    lo, hi = _matmul_halves(x, W)

    per_tile = -(-n_edges // (N_TILES * CHUNK)) * CHUNK
    e_pad = per_tile * N_TILES
    pad = e_pad - n_edges
    src = jnp.pad(edge_index[0].astype(jnp.int32), (0, pad))
    dst = jnp.pad(edge_index[1].astype(jnp.int32), (0, pad))
    ew = jnp.pad(edge_weight.astype(jnp.float32), (0, pad))
    n_chunks = per_tile // CHUNK
    src3 = src.reshape(N_TILES, n_chunks, CHUNK)
    dst3 = dst.reshape(N_TILES, n_chunks, CHUNK)
    ew3 = ew.reshape(N_TILES, n_chunks, CHUNK)

    n_pad = -(-n_nodes // (N_TILES * CHUNK)) * (N_TILES * CHUNK)
    sc_agg = _make_sc_agg(n_pad, half, n_chunks)
    out_lo, out_hi = sc_agg(src3, dst3, ew3, lo, hi)
    return jnp.concatenate([out_lo[:n_nodes], out_hi[:n_nodes]], axis=1) + b
